# baseline (speedup 1.0000x reference)
import jax
import jax.numpy as jnp
from jax.experimental import pallas as pl
from jax.experimental.pallas import tpu as pltpu

_P = 64
_CH = 8
_LOG2E = 1.4426950408889634
_LN2 = 0.6931471805599453


def _loss_kernel(eps0_ref, eps1_ref, aux_ref, out_ref):
    t = eps0_ref.shape[1]
    nb = eps0_ref.shape[2]
    steps = t // _CH
    rem = t - steps * _CH

    def task(eps_ref, base):
        l0 = jnp.broadcast_to(aux_ref[base + 0:base + 1, :], (_CH, nb))
        l1 = jnp.broadcast_to(aux_ref[base + 1:base + 2, :], (_CH, nb))
        l2 = jnp.broadcast_to(aux_ref[base + 2:base + 3, :], (_CH, nb))
        s2 = jnp.broadcast_to(aux_ref[base + 3:base + 4, :], (_CH, nb))

        def slab(off, rows, accs):
            x0 = eps_ref[0, pl.ds(off, rows), :]
            x1 = eps_ref[1, pl.ds(off, rows), :]
            x2 = eps_ref[2, pl.ds(off, rows), :]
            e = (jnp.exp2(l0[:rows] + x0 * s2[:rows])
                 + jnp.exp2(l1[:rows] + x1 * s2[:rows])
                 + jnp.exp2(l2[:rows] + x2 * s2[:rows]))
            lg = jnp.log(jnp.maximum(e, 1e-30))
            aL, aX0, aX1, aX2 = accs
            return aL + lg, aX0 + x0, aX1 + x1, aX2 + x2

        def body(i, carry):
            return slab(pl.multiple_of(i * _CH, _CH), _CH, carry)

        zeros = jnp.zeros((_CH, nb), jnp.float32)
        accs = jax.lax.fori_loop(0, steps, body, (zeros,) * 4)
        aL, aX0, aX1, aX2 = (jnp.sum(a, axis=0, keepdims=True) for a in accs)
        if rem:
            z = jnp.zeros((rem, nb), jnp.float32)
            eaccs = slab(steps * _CH, rem, (z,) * 4)
            eL, eX0, eX1, eX2 = (
                jnp.sum(a, axis=0, keepdims=True) for a in eaccs)
            aL, aX0, aX1, aX2 = aL + eL, aX0 + eX0, aX1 + eX1, aX2 + eX2
        w0 = aux_ref[base + 4:base + 5, :]
        w1 = aux_ref[base + 5:base + 6, :]
        w2 = aux_ref[base + 6:base + 7, :]
        yt = aux_ref[base + 7:base + 8, :]
        sc = aux_ref[base + 8:base + 9, :]
        tdotwl = aux_ref[base + 9:base + 10, :]
        return yt * aL - tdotwl - sc * (w0 * aX0 + w1 * aX1 + w2 * aX2)

    out_ref[0] = jnp.concatenate(
        [task(eps0_ref, 0), task(eps1_ref, 12)], axis=0)


def _aux_rows(y_true, y_pred, t):
    lg = y_pred[:, :3].T
    sc = jnp.exp(0.5 * y_pred[:, 3])[None, :]
    w = y_true.T
    yt = jnp.sum(y_true, axis=1)[None, :]
    tdotwl = t * jnp.sum(w * lg, axis=0, keepdims=True)
    z = jnp.zeros_like(sc)
    return jnp.concatenate(
        [lg * _LOG2E, sc * _LOG2E, w, yt, sc, tdotwl, z, z], axis=0)


def kernel(y_true0, y_pred0, y_true1, y_pred1, log_vars, eps0, eps1):
    t, n, _ = eps0.shape
    nb = n // _P

    e0 = jnp.transpose(eps0, (2, 0, 1))
    e1 = jnp.transpose(eps1, (2, 0, 1))
    aux = jnp.concatenate(
        [_aux_rows(y_true0, y_pred0, t), _aux_rows(y_true1, y_pred1, t)],
        axis=0)

    out = pl.pallas_call(
        _loss_kernel,
        grid=(_P,),
        in_specs=[
            pl.BlockSpec((3, t, nb), lambda p: (0, 0, p)),
            pl.BlockSpec((3, t, nb), lambda p: (0, 0, p)),
            pl.BlockSpec((24, nb), lambda p: (0, p)),
        ],
        out_specs=pl.BlockSpec((1, 2, nb), lambda p: (p, 0, 0)),
        out_shape=jax.ShapeDtypeStruct((_P, 2, nb), jnp.float32),
        compiler_params=pltpu.CompilerParams(
            dimension_semantics=("parallel",)),
    )(e0, e1, aux)

    inv_tn = 1.0 / (t * n)
    mc0 = jnp.sum(out[:, 0, :]) * inv_tn
    mc1 = jnp.sum(out[:, 1, :]) * inv_tn
    lv0, lv1 = log_vars[0], log_vars[1]
    return jnp.exp(-lv0) * mc0 + lv0 + jnp.exp(-lv1) * mc1 + lv1


# speedup vs baseline: 1.1765x; 1.1765x over previous
import jax
import jax.numpy as jnp
from jax.experimental import pallas as pl
from jax.experimental.pallas import tpu as pltpu

_P = 64
_CH = 8
_LOG2E = 1.4426950408889634
_LN2 = 0.6931471805599453


def _loss_kernel(eps0_ref, eps1_ref, aux_ref, out_ref):
    t = eps0_ref.shape[1]
    nb = eps0_ref.shape[2]
    steps = t // _CH
    rem = t - steps * _CH

    def slab(eps_ref, base, off, rows, accs):
        l0 = aux_ref[base + 0:base + 1, :]
        l1 = aux_ref[base + 1:base + 2, :]
        l2 = aux_ref[base + 2:base + 3, :]
        s2 = aux_ref[base + 3:base + 4, :]
        w0 = aux_ref[base + 4:base + 5, :]
        w1 = aux_ref[base + 5:base + 6, :]
        w2 = aux_ref[base + 6:base + 7, :]
        x0 = eps_ref[0, pl.ds(off, rows), :]
        x1 = eps_ref[1, pl.ds(off, rows), :]
        x2 = eps_ref[2, pl.ds(off, rows), :]
        e = (jnp.exp2(l0 + x0 * s2) + jnp.exp2(l1 + x1 * s2)
             + jnp.exp2(l2 + x2 * s2))
        lg = jnp.log(jnp.maximum(e, 1e-30))
        aL, aWX = accs
        return aL + lg, aWX + (w0 * x0 + w1 * x1 + w2 * x2)

    def body(i, carry):
        a0, a1 = carry
        off = pl.multiple_of(i * _CH, _CH)
        return (slab(eps0_ref, 0, off, _CH, a0),
                slab(eps1_ref, 12, off, _CH, a1))

    zeros = jnp.zeros((_CH, nb), jnp.float32)
    acc0, acc1 = jax.lax.fori_loop(
        0, steps, body, ((zeros, zeros), (zeros, zeros)))

    def finalize(eps_ref, base, accs):
        aL = jnp.sum(accs[0], axis=0, keepdims=True)
        aWX = jnp.sum(accs[1], axis=0, keepdims=True)
        if rem:
            z = jnp.zeros((rem, nb), jnp.float32)
            eL, eWX = slab(eps_ref, base, steps * _CH, rem, (z, z))
            aL = aL + jnp.sum(eL, axis=0, keepdims=True)
            aWX = aWX + jnp.sum(eWX, axis=0, keepdims=True)
        yt = aux_ref[base + 7:base + 8, :]
        sc = aux_ref[base + 8:base + 9, :]
        tdotwl = aux_ref[base + 9:base + 10, :]
        return yt * aL - tdotwl - sc * aWX

    out_ref[0] = jnp.concatenate(
        [finalize(eps0_ref, 0, acc0), finalize(eps1_ref, 12, acc1)], axis=0)


def _aux_rows(y_true, y_pred, t):
    lg = y_pred[:, :3].T
    sc = jnp.exp(0.5 * y_pred[:, 3])[None, :]
    w = y_true.T
    yt = jnp.sum(y_true, axis=1)[None, :]
    tdotwl = t * jnp.sum(w * lg, axis=0, keepdims=True)
    z = jnp.zeros_like(sc)
    return jnp.concatenate(
        [lg * _LOG2E, sc * _LOG2E, w, yt, sc, tdotwl, z, z], axis=0)


def kernel(y_true0, y_pred0, y_true1, y_pred1, log_vars, eps0, eps1):
    t, n, _ = eps0.shape
    nb = n // _P

    e0 = jnp.transpose(eps0, (2, 0, 1))
    e1 = jnp.transpose(eps1, (2, 0, 1))
    aux = jnp.concatenate(
        [_aux_rows(y_true0, y_pred0, t), _aux_rows(y_true1, y_pred1, t)],
        axis=0)

    out = pl.pallas_call(
        _loss_kernel,
        grid=(_P,),
        in_specs=[
            pl.BlockSpec((3, t, nb), lambda p: (0, 0, p)),
            pl.BlockSpec((3, t, nb), lambda p: (0, 0, p)),
            pl.BlockSpec((24, nb), lambda p: (0, p)),
        ],
        out_specs=pl.BlockSpec((1, 2, nb), lambda p: (p, 0, 0)),
        out_shape=jax.ShapeDtypeStruct((_P, 2, nb), jnp.float32),
        compiler_params=pltpu.CompilerParams(
            dimension_semantics=("arbitrary",)),
    )(e0, e1, aux)

    inv_tn = 1.0 / (t * n)
    mc0 = jnp.sum(out[:, 0, :]) * inv_tn
    mc1 = jnp.sum(out[:, 1, :]) * inv_tn
    lv0, lv1 = log_vars[0], log_vars[1]
    return jnp.exp(-lv0) * mc0 + lv0 + jnp.exp(-lv1) * mc1 + lv1


# speedup vs baseline: 1.2716x; 1.0808x over previous
import jax
import jax.numpy as jnp
from jax.experimental import pallas as pl
from jax.experimental.pallas import tpu as pltpu

_P = 64
_CH = 8
_LOG2E = 1.4426950408889634


def _loss_kernel(eps0_ref, eps1_ref, auxb_ref, aux_ref, out_ref):
    t = eps0_ref.shape[1]
    nb = eps0_ref.shape[2]
    steps = t // _CH
    rem = t - steps * _CH

    def slab(eps_ref, base, off, rows, accs):
        l0 = auxb_ref[base + 0, :rows, :]
        l1 = auxb_ref[base + 1, :rows, :]
        l2 = auxb_ref[base + 2, :rows, :]
        s2 = auxb_ref[base + 3, :rows, :]
        w0 = auxb_ref[base + 4, :rows, :]
        w1 = auxb_ref[base + 5, :rows, :]
        w2 = auxb_ref[base + 6, :rows, :]
        x0 = eps_ref[0, pl.ds(off, rows), :]
        x1 = eps_ref[1, pl.ds(off, rows), :]
        x2 = eps_ref[2, pl.ds(off, rows), :]
        e = (jnp.exp2(l0 + x0 * s2) + jnp.exp2(l1 + x1 * s2)
             + jnp.exp2(l2 + x2 * s2))
        lg = jnp.log(jnp.maximum(e, 1e-30))
        aL, aWX = accs
        return aL + lg, aWX + (w0 * x0 + w1 * x1 + w2 * x2)

    def body(i, carry):
        a0, a1 = carry
        off = pl.multiple_of(i * (2 * _CH), 2 * _CH)
        a0 = slab(eps0_ref, 0, off, _CH, a0)
        a1 = slab(eps1_ref, 7, off, _CH, a1)
        a0 = slab(eps0_ref, 0, off + _CH, _CH, a0)
        a1 = slab(eps1_ref, 7, off + _CH, _CH, a1)
        return a0, a1

    zeros = jnp.zeros((_CH, nb), jnp.float32)
    acc0, acc1 = jax.lax.fori_loop(
        0, steps // 2, body, ((zeros, zeros), (zeros, zeros)))

    def finalize(eps_ref, base, abase, accs):
        aL = jnp.sum(accs[0], axis=0, keepdims=True)
        aWX = jnp.sum(accs[1], axis=0, keepdims=True)
        if rem:
            z = jnp.zeros((rem, nb), jnp.float32)
            eL, eWX = slab(eps_ref, abase, steps * _CH, rem, (z, z))
            aL = aL + jnp.sum(eL, axis=0, keepdims=True)
            aWX = aWX + jnp.sum(eWX, axis=0, keepdims=True)
        yt = aux_ref[base + 0:base + 1, :]
        sc = aux_ref[base + 1:base + 2, :]
        tdotwl = aux_ref[base + 2:base + 3, :]
        return yt * aL - tdotwl - sc * aWX

    out_ref[0] = jnp.concatenate(
        [finalize(eps0_ref, 0, 0, acc0), finalize(eps1_ref, 4, 7, acc1)],
        axis=0)


def _aux_parts(y_true, y_pred, t):
    lg = y_pred[:, :3].T
    sc = jnp.exp(0.5 * y_pred[:, 3])[None, :]
    w = y_true.T
    yt = jnp.sum(y_true, axis=1)[None, :]
    tdotwl = t * jnp.sum(w * lg, axis=0, keepdims=True)
    loop_rows = jnp.concatenate([lg * _LOG2E, sc * _LOG2E, w], axis=0)
    fin_rows = jnp.concatenate([yt, sc, tdotwl, jnp.zeros_like(sc)], axis=0)
    return loop_rows, fin_rows


def kernel(y_true0, y_pred0, y_true1, y_pred1, log_vars, eps0, eps1):
    t, n, _ = eps0.shape
    nb = n // _P

    e0 = jnp.transpose(eps0, (2, 0, 1))
    e1 = jnp.transpose(eps1, (2, 0, 1))
    loop0, fin0 = _aux_parts(y_true0, y_pred0, t)
    loop1, fin1 = _aux_parts(y_true1, y_pred1, t)
    auxb = jnp.broadcast_to(
        jnp.concatenate([loop0, loop1], axis=0)[:, None, :], (14, _CH, n))
    aux = jnp.concatenate([fin0, fin1], axis=0)

    out = pl.pallas_call(
        _loss_kernel,
        grid=(_P,),
        in_specs=[
            pl.BlockSpec((3, t, nb), lambda p: (0, 0, p)),
            pl.BlockSpec((3, t, nb), lambda p: (0, 0, p)),
            pl.BlockSpec((14, _CH, nb), lambda p: (0, 0, p)),
            pl.BlockSpec((8, nb), lambda p: (0, p)),
        ],
        out_specs=pl.BlockSpec((1, 2, nb), lambda p: (p, 0, 0)),
        out_shape=jax.ShapeDtypeStruct((_P, 2, nb), jnp.float32),
        compiler_params=pltpu.CompilerParams(
            dimension_semantics=("arbitrary",)),
    )(e0, e1, auxb, aux)

    inv_tn = 1.0 / (t * n)
    mc0 = jnp.sum(out[:, 0, :]) * inv_tn
    mc1 = jnp.sum(out[:, 1, :]) * inv_tn
    lv0, lv1 = log_vars[0], log_vars[1]
    return jnp.exp(-lv0) * mc0 + lv0 + jnp.exp(-lv1) * mc1 + lv1


# speedup vs baseline: 1.4104x; 1.1091x over previous
import jax
import jax.numpy as jnp
from jax.experimental import pallas as pl
from jax.experimental.pallas import tpu as pltpu

_P = 16
_LC = 512
_CH = 8
_LOG2E = 1.4426950408889634


def _loss_kernel(eps0_ref, eps1_ref, auxb_ref, aux_ref, out_ref):
    t = eps0_ref.shape[1]
    nb = eps0_ref.shape[2]
    steps = t // _CH
    rem = t - steps * _CH

    def slab(eps_ref, base, off, rows, lo, accs):
        l0 = auxb_ref[base + 0, :rows, lo:lo + _LC]
        l1 = auxb_ref[base + 1, :rows, lo:lo + _LC]
        l2 = auxb_ref[base + 2, :rows, lo:lo + _LC]
        s2 = auxb_ref[base + 3, :rows, lo:lo + _LC]
        w0 = auxb_ref[base + 4, :rows, lo:lo + _LC]
        w1 = auxb_ref[base + 5, :rows, lo:lo + _LC]
        w2 = auxb_ref[base + 6, :rows, lo:lo + _LC]
        x0 = eps_ref[0, pl.ds(off, rows), lo:lo + _LC]
        x1 = eps_ref[1, pl.ds(off, rows), lo:lo + _LC]
        x2 = eps_ref[2, pl.ds(off, rows), lo:lo + _LC]
        e = (jnp.exp2(l0 + x0 * s2) + jnp.exp2(l1 + x1 * s2)
             + jnp.exp2(l2 + x2 * s2))
        lg = jnp.log(jnp.maximum(e, 1e-30))
        aL, aWX = accs
        return aL + lg, aWX + (w0 * x0 + w1 * x1 + w2 * x2)

    def chunk(lo):
        def body(i, carry):
            a0, a1 = carry
            off = pl.multiple_of(i * (2 * _CH), 2 * _CH)
            a0 = slab(eps0_ref, 0, off, _CH, lo, a0)
            a1 = slab(eps1_ref, 7, off, _CH, lo, a1)
            a0 = slab(eps0_ref, 0, off + _CH, _CH, lo, a0)
            a1 = slab(eps1_ref, 7, off + _CH, _CH, lo, a1)
            return a0, a1

        zeros = jnp.zeros((_CH, _LC), jnp.float32)
        acc0, acc1 = jax.lax.fori_loop(
            0, steps // 2, body, ((zeros, zeros), (zeros, zeros)))

        def finalize(eps_ref, base, abase, accs):
            aL = jnp.sum(accs[0], axis=0, keepdims=True)
            aWX = jnp.sum(accs[1], axis=0, keepdims=True)
            if rem:
                z = jnp.zeros((rem, _LC), jnp.float32)
                eL, eWX = slab(eps_ref, abase, steps * _CH, rem, lo, (z, z))
                aL = aL + jnp.sum(eL, axis=0, keepdims=True)
                aWX = aWX + jnp.sum(eWX, axis=0, keepdims=True)
            yt = aux_ref[base + 0:base + 1, lo:lo + _LC]
            sc = aux_ref[base + 1:base + 2, lo:lo + _LC]
            tdotwl = aux_ref[base + 2:base + 3, lo:lo + _LC]
            return yt * aL - tdotwl - sc * aWX

        out_ref[0, 0:1, lo:lo + _LC] = finalize(eps0_ref, 0, 0, acc0)
        out_ref[0, 1:2, lo:lo + _LC] = finalize(eps1_ref, 4, 7, acc1)

    for j in range(nb // _LC):
        chunk(j * _LC)


def _aux_parts(y_true, y_pred, t):
    lg = y_pred[:, :3].T
    sc = jnp.exp(0.5 * y_pred[:, 3])[None, :]
    w = y_true.T
    yt = jnp.sum(y_true, axis=1)[None, :]
    tdotwl = t * jnp.sum(w * lg, axis=0, keepdims=True)
    loop_rows = jnp.concatenate([lg * _LOG2E, sc * _LOG2E, w], axis=0)
    fin_rows = jnp.concatenate([yt, sc, tdotwl, jnp.zeros_like(sc)], axis=0)
    return loop_rows, fin_rows


def kernel(y_true0, y_pred0, y_true1, y_pred1, log_vars, eps0, eps1):
    t, n, _ = eps0.shape
    nb = n // _P

    e0 = jnp.transpose(eps0, (2, 0, 1))
    e1 = jnp.transpose(eps1, (2, 0, 1))
    loop0, fin0 = _aux_parts(y_true0, y_pred0, t)
    loop1, fin1 = _aux_parts(y_true1, y_pred1, t)
    auxb = jnp.broadcast_to(
        jnp.concatenate([loop0, loop1], axis=0)[:, None, :], (14, _CH, n))
    aux = jnp.concatenate([fin0, fin1], axis=0)

    out = pl.pallas_call(
        _loss_kernel,
        grid=(_P,),
        in_specs=[
            pl.BlockSpec((3, t, nb), lambda p: (0, 0, p)),
            pl.BlockSpec((3, t, nb), lambda p: (0, 0, p)),
            pl.BlockSpec((14, _CH, nb), lambda p: (0, 0, p)),
            pl.BlockSpec((8, nb), lambda p: (0, p)),
        ],
        out_specs=pl.BlockSpec((1, 2, nb), lambda p: (p, 0, 0)),
        out_shape=jax.ShapeDtypeStruct((_P, 2, nb), jnp.float32),
        compiler_params=pltpu.CompilerParams(
            dimension_semantics=("arbitrary",),
            vmem_limit_bytes=60 * 1024 * 1024),
    )(e0, e1, auxb, aux)

    inv_tn = 1.0 / (t * n)
    mc0 = jnp.sum(out[:, 0, :]) * inv_tn
    mc1 = jnp.sum(out[:, 1, :]) * inv_tn
    lv0, lv1 = log_vars[0], log_vars[1]
    return jnp.exp(-lv0) * mc0 + lv0 + jnp.exp(-lv1) * mc1 + lv1
